# Initial kernel scaffold; baseline (speedup 1.0000x reference)
import jax
import jax.numpy as jnp
from jax.experimental import pallas as pl


def kernel(x, W):
    raise NotImplementedError("write your pallas kernel here")


# speedup vs baseline: 1.4418x; 1.4418x over previous
import functools
import math

import jax
import jax.numpy as jnp
from jax import lax
from jax.experimental import pallas as pl
from jax.experimental.pallas import tpu as pltpu
from jax.experimental.pallas import tpu_sc as plsc

B = 4096
L = 200
D = 128
V = 100000

CH = 100
NB = B * L // CH
NW = 32
CPW = NB // NW


def _pos_body(o_ref):
    pos = lax.broadcasted_iota(jnp.int32, (L, D), 0).astype(jnp.float32)
    col = lax.broadcasted_iota(jnp.int32, (L, D), 1)
    k2 = col - (col % 2)
    div = jnp.exp(k2.astype(jnp.float32) * (-math.log(10000.0) / D))
    ang = pos * div
    o_ref[...] = jnp.where(col % 2 == 0, jnp.sin(ang), jnp.cos(ang))


_pos_call = pl.pallas_call(
    _pos_body, out_shape=jax.ShapeDtypeStruct((L, D), jnp.float32)
)

_mesh = plsc.VectorSubcoreMesh(core_axis_name="c", subcore_axis_name="s")


@functools.partial(
    pl.kernel,
    mesh=_mesh,
    out_type=jax.ShapeDtypeStruct((NB, CH, D), jnp.float32),
    scratch_types=[
        pltpu.VMEM((CH,), jnp.int32),
        pltpu.VMEM((CH, D), jnp.float32),
        pltpu.VMEM((2, CH, D), jnp.float32),
        pltpu.SemaphoreType.DMA,
    ],
)
def _sc_embed(x_hbm, w_hbm, pos_hbm, out_hbm, idx_v, rows_v, pos_v, sem):
    wid = lax.axis_index("s") * 2 + lax.axis_index("c")
    base = wid * CPW
    pltpu.sync_copy(pos_hbm, pos_v)

    def chunk(i, carry):
        j = base + i
        pltpu.sync_copy(x_hbm.at[j], idx_v)
        pltpu.async_copy(w_hbm.at[idx_v], rows_v, sem).wait()
        p = i % 2

        def row(r, c2):
            for k in range(D // 16):
                rows_v[r, pl.ds(k * 16, 16)] += pos_v[p, r, pl.ds(k * 16, 16)]
            return c2

        lax.fori_loop(0, CH, row, 0)
        pltpu.sync_copy(rows_v, out_hbm.at[j])
        return carry

    lax.fori_loop(0, CPW, chunk, 0)


def kernel(x, W):
    pos = _pos_call().reshape(2, CH, D)
    x2 = x.astype(jnp.int32).reshape(NB, CH)
    out = _sc_embed(x2, W, pos)
    return out.reshape(B, L, D)


# speedup vs baseline: 3.6543x; 2.5345x over previous
import functools
import math

import jax
import jax.numpy as jnp
from jax import lax
from jax.experimental import pallas as pl
from jax.experimental.pallas import tpu as pltpu
from jax.experimental.pallas import tpu_sc as plsc

B = 4096
L = 200
D = 128
V = 100000

CH = 100
NB = B * L // CH
NW = 32
CPW = NB // NW
NBUF = 4
LOOKAHEAD = 2


def _pos_body(o_ref):
    pos = lax.broadcasted_iota(jnp.int32, (L, D), 0).astype(jnp.float32)
    col = lax.broadcasted_iota(jnp.int32, (L, D), 1)
    k2 = col - (col % 2)
    div = jnp.exp(k2.astype(jnp.float32) * (-math.log(10000.0) / D))
    ang = pos * div
    o_ref[...] = jnp.where(col % 2 == 0, jnp.sin(ang), jnp.cos(ang))


_pos_call = pl.pallas_call(
    _pos_body, out_shape=jax.ShapeDtypeStruct((L, D), jnp.float32)
)

_mesh = plsc.VectorSubcoreMesh(core_axis_name="c", subcore_axis_name="s")


@functools.partial(
    pl.kernel,
    mesh=_mesh,
    out_type=jax.ShapeDtypeStruct((NB, CH, D), jnp.float32),
    scratch_types=(
        [pltpu.VMEM((CH,), jnp.int32) for _ in range(NBUF)]
        + [pltpu.VMEM((CH, D), jnp.float32) for _ in range(NBUF)]
        + [pltpu.VMEM((2, CH, D), jnp.float32)]
        + [pltpu.SemaphoreType.DMA for _ in range(2 * NBUF)]
    ),
)
def _sc_embed(x_hbm, w_hbm, pos_hbm, out_hbm, *refs):
    idxs = refs[0:NBUF]
    rows = refs[NBUF:2 * NBUF]
    pos_v = refs[2 * NBUF]
    gsem = refs[2 * NBUF + 1:3 * NBUF + 1]
    osem = refs[3 * NBUF + 1:4 * NBUF + 1]

    wid = lax.axis_index("s") * 2 + lax.axis_index("c")
    base = wid * CPW
    pltpu.sync_copy(pos_hbm, pos_v)

    for b in range(LOOKAHEAD):
        pltpu.sync_copy(x_hbm.at[base + b], idxs[b])
        pltpu.async_copy(w_hbm.at[idxs[b]], rows[b], gsem[b])

    def step(t, carry):
        for b in range(NBUF):
            i = t * NBUF + b
            pltpu.make_async_copy(w_hbm.at[idxs[b]], rows[b], gsem[b]).wait()
            p = b & 1

            def row(r, c2, _b=b, _p=p):
                for k in range(D // 16):
                    rows[_b][r, pl.ds(k * 16, 16)] += (
                        pos_v[_p, r, pl.ds(k * 16, 16)]
                    )
                return c2

            lax.fori_loop(0, CH, row, 0)
            pltpu.async_copy(rows[b], out_hbm.at[base + i], osem[b])

            nxt = i + LOOKAHEAD
            b2 = (b + LOOKAHEAD) % NBUF

            @pl.when(jnp.logical_and(nxt >= NBUF, nxt < CPW))
            def _drain(_b2=b2):
                pltpu.make_async_copy(
                    rows[_b2], out_hbm.at[base], osem[_b2]
                ).wait()

            @pl.when(nxt < CPW)
            def _prefetch(_b2=b2, _nxt=nxt):
                pltpu.sync_copy(x_hbm.at[base + _nxt], idxs[_b2])
                pltpu.async_copy(w_hbm.at[idxs[_b2]], rows[_b2], gsem[_b2])

        return carry

    lax.fori_loop(0, CPW // NBUF, step, 0)
    for b in range(NBUF):
        pltpu.make_async_copy(rows[b], out_hbm.at[base], osem[b]).wait()


def kernel(x, W):
    pos = _pos_call().reshape(2, CH, D)
    x2 = x.astype(jnp.int32).reshape(NB, CH)
    out = _sc_embed(x2, W, pos)
    return out.reshape(B, L, D)


# speedup vs baseline: 3.9587x; 1.0833x over previous
import functools
import math

import jax
import jax.numpy as jnp
from jax import lax
from jax.experimental import pallas as pl
from jax.experimental.pallas import tpu as pltpu
from jax.experimental.pallas import tpu_sc as plsc

B = 4096
L = 200
D = 128
V = 100000

CH = 100
NB = B * L // CH
NW = 32
CPW = NB // NW
NBUF = 4
LOOKAHEAD = 2


def _pos_body(o_ref):
    pos = lax.broadcasted_iota(jnp.int32, (L, D), 0).astype(jnp.float32)
    col = lax.broadcasted_iota(jnp.int32, (L, D), 1)
    k2 = col - (col % 2)
    div = jnp.exp(k2.astype(jnp.float32) * (-math.log(10000.0) / D))
    ang = pos * div
    o_ref[...] = jnp.where(col % 2 == 0, jnp.sin(ang), jnp.cos(ang))


_pos_call = pl.pallas_call(
    _pos_body, out_shape=jax.ShapeDtypeStruct((L, D), jnp.float32)
)

_mesh = plsc.VectorSubcoreMesh(core_axis_name="c", subcore_axis_name="s")


@functools.partial(
    pl.kernel,
    mesh=_mesh,
    out_type=jax.ShapeDtypeStruct((NB, CH, D), jnp.float32),
    scratch_types=(
        [pltpu.VMEM((CPW, CH), jnp.int32)]
        + [pltpu.VMEM((CH, D), jnp.float32) for _ in range(NBUF)]
        + [pltpu.VMEM((2, CH, D), jnp.float32)]
        + [pltpu.SemaphoreType.DMA for _ in range(2 * NBUF)]
    ),
)
def _sc_embed(x_hbm, w_hbm, pos_hbm, out_hbm, *refs):
    idx_v = refs[0]
    rows = refs[1:NBUF + 1]
    pos_v = refs[NBUF + 1]
    gsem = refs[NBUF + 2:2 * NBUF + 2]
    osem = refs[2 * NBUF + 2:3 * NBUF + 2]

    wid = lax.axis_index("s") * 2 + lax.axis_index("c")
    base = wid * CPW
    pltpu.sync_copy(x_hbm.at[pl.ds(base, CPW)], idx_v)
    pltpu.sync_copy(pos_hbm, pos_v)

    for b in range(LOOKAHEAD):
        pltpu.async_copy(w_hbm.at[idx_v.at[b]], rows[b], gsem[b])

    def step(t, carry):
        for b in range(NBUF):
            i = t * NBUF + b
            pltpu.make_async_copy(
                w_hbm.at[idx_v.at[i]], rows[b], gsem[b]
            ).wait()
            p = b & 1

            def row(r, c2, _b=b, _p=p):
                for k in range(D // 16):
                    plsc.addupdate(
                        rows[_b].at[r, pl.ds(k * 16, 16)],
                        pos_v[_p, r, pl.ds(k * 16, 16)],
                    )
                return c2

            lax.fori_loop(0, CH, row, 0)
            pltpu.async_copy(rows[b], out_hbm.at[base + i], osem[b])

            nxt = i + LOOKAHEAD
            b2 = (b + LOOKAHEAD) % NBUF

            @pl.when(jnp.logical_and(nxt >= NBUF, nxt < CPW))
            def _drain(_b2=b2):
                pltpu.make_async_copy(
                    rows[_b2], out_hbm.at[base], osem[_b2]
                ).wait()

            @pl.when(nxt < CPW)
            def _prefetch(_b2=b2, _nxt=nxt):
                pltpu.async_copy(
                    w_hbm.at[idx_v.at[_nxt]], rows[_b2], gsem[_b2]
                )

        return carry

    lax.fori_loop(0, CPW // NBUF, step, 0)
    for b in range(NBUF):
        pltpu.make_async_copy(rows[b], out_hbm.at[base], osem[b]).wait()


def kernel(x, W):
    pos = _pos_call().reshape(2, CH, D)
    x2 = x.astype(jnp.int32).reshape(NB, CH)
    out = _sc_embed(x2, W, pos)
    return out.reshape(B, L, D)


# speedup vs baseline: 5.8054x; 1.4665x over previous
import functools
import math

import jax
import jax.numpy as jnp
from jax import lax
from jax.experimental import pallas as pl
from jax.experimental.pallas import tpu as pltpu
from jax.experimental.pallas import tpu_sc as plsc

B = 4096
L = 200
D = 128
V = 100000

CH = 40
PER_SEQ = L // CH
NB = B * L // CH
NW = 32
CPW = NB // NW
SPW = B // NW
NBUF = PER_SEQ
LOOKAHEAD = 2


def _pos_body(o_ref):
    pos = lax.broadcasted_iota(jnp.int32, (L, D), 0).astype(jnp.float32)
    col = lax.broadcasted_iota(jnp.int32, (L, D), 1)
    k2 = col - (col % 2)
    div = jnp.exp(k2.astype(jnp.float32) * (-math.log(10000.0) / D))
    ang = pos * div
    o_ref[...] = jnp.where(col % 2 == 0, jnp.sin(ang), jnp.cos(ang))


_pos_call = pl.pallas_call(
    _pos_body, out_shape=jax.ShapeDtypeStruct((L, D), jnp.float32)
)

_mesh = plsc.VectorSubcoreMesh(core_axis_name="c", subcore_axis_name="s")


@functools.partial(
    pl.kernel,
    mesh=_mesh,
    out_type=jax.ShapeDtypeStruct((B, L, D), jnp.float32),
    scratch_types=(
        [pltpu.VMEM((CPW // 2, CH), jnp.int32)]
        + [pltpu.VMEM((CH, D), jnp.float32) for _ in range(NBUF)]
        + [pltpu.VMEM((L, D), jnp.float32)]
        + [pltpu.SemaphoreType.DMA for _ in range(2 * NBUF)]
    ),
)
def _sc_embed(x_hbm, w_hbm, pos_hbm, out_hbm, *refs):
    idx_v = refs[0]
    rows = refs[1:NBUF + 1]
    pos_v = refs[NBUF + 1]
    gsem = refs[NBUF + 2:2 * NBUF + 2]
    osem = refs[2 * NBUF + 2:3 * NBUF + 2]

    CPH = CPW // 2
    TPH = CPH // NBUF

    wid = lax.axis_index("s") * 2 + lax.axis_index("c")
    base = wid * CPW
    pltpu.sync_copy(pos_hbm, pos_v)

    for q in range(2):
        pltpu.sync_copy(x_hbm.at[pl.ds(base + q * CPH, CPH)], idx_v)

        for b in range(LOOKAHEAD):
            pltpu.async_copy(w_hbm.at[idx_v.at[b]], rows[b], gsem[b])

        def step(t, carry, _q=q):
            for b in range(NBUF):
                i = t * NBUF + b
                pltpu.make_async_copy(
                    w_hbm.at[idx_v.at[i]], rows[b], gsem[b]
                ).wait()
                poff = b * CH

                def row(r, c2, _b=b, _poff=poff):
                    for u in range(2):
                        for k in range(D // 16):
                            plsc.addupdate(
                                rows[_b].at[2 * r + u, pl.ds(k * 16, 16)],
                                pos_v[_poff + 2 * r + u, pl.ds(k * 16, 16)],
                            )
                    return c2

                lax.fori_loop(0, CH // 2, row, 0)
                pltpu.async_copy(
                    rows[b],
                    out_hbm.at[
                        wid * SPW + _q * TPH + t, pl.ds(b * CH, CH)
                    ],
                    osem[b],
                )

                nxt = i + LOOKAHEAD
                b2 = (b + LOOKAHEAD) % NBUF

                @pl.when(jnp.logical_and(nxt >= NBUF, nxt < CPH))
                def _drain(_b2=b2):
                    pltpu.make_async_copy(
                        rows[_b2], out_hbm.at[0, pl.ds(0, CH)], osem[_b2]
                    ).wait()

                @pl.when(nxt < CPH)
                def _prefetch(_b2=b2, _nxt=nxt):
                    pltpu.async_copy(
                        w_hbm.at[idx_v.at[_nxt]], rows[_b2], gsem[_b2]
                    )

            return carry

        lax.fori_loop(0, TPH, step, 0)
        for b in range(NBUF):
            pltpu.make_async_copy(
                rows[b], out_hbm.at[0, pl.ds(0, CH)], osem[b]
            ).wait()


def kernel(x, W):
    pos = _pos_call()
    x2 = x.astype(jnp.int32).reshape(NB, CH)
    return _sc_embed(x2, W, pos)
